# Initial kernel scaffold; baseline (speedup 1.0000x reference)
import jax
import jax.numpy as jnp
from jax.experimental import pallas as pl


def kernel(output, target):
    raise NotImplementedError("write your pallas kernel here")


# speedup vs baseline: 3.8982x; 3.8982x over previous
import jax
import jax.numpy as jnp
from jax.experimental import pallas as pl
from jax.experimental.pallas import tpu as pltpu

NBINS = 256
H = 512
W = 512
ROWS_PER_STEP = 8


def _iou_kernel(x_ref, t_ref, o_ref, p_ref, acc_ref):
    p_ref[...] = jax.nn.sigmoid(x_ref[0])
    p = p_ref[...]
    mn = jnp.min(p)
    mx = jnp.max(p)
    span = mx - mn
    scale = jnp.where(span > 0, jnp.float32(NBINS) / span, jnp.float32(0.0))

    acc_ref[...] = jnp.zeros_like(acc_ref)
    bins3 = jax.lax.broadcasted_iota(jnp.int32, (NBINS, 1, 1), 0).astype(
        jnp.float32)

    def hbody(r, carry):
        rows = p_ref[pl.ds(r * ROWS_PER_STEP, ROWS_PER_STEP), :]
        idx = jnp.clip(jnp.floor((rows - mn) * scale), 0.0, float(NBINS - 1))
        onehot = jnp.where(idx[None, :, :] == bins3, 1.0, 0.0)
        acc_ref[...] += jnp.sum(onehot, axis=1)
        return carry

    jax.lax.fori_loop(0, H // ROWS_PER_STEP, hbody, 0)

    hist = jnp.sum(acc_ref[...], axis=1, keepdims=True)
    bins = jax.lax.broadcasted_iota(jnp.int32, (NBINS, 1), 0).astype(
        jnp.float32)
    centers = mn + (bins + 0.5) * (span / NBINS)
    rhs = jnp.concatenate([hist, hist * centers], axis=1)
    ri = jax.lax.broadcasted_iota(jnp.int32, (NBINS, NBINS), 0)
    ci = jax.lax.broadcasted_iota(jnp.int32, (NBINS, NBINS), 1)
    ltri = jnp.where(ci <= ri, 1.0, 0.0)
    cum = jax.lax.dot_general(
        ltri, rhs, (((1,), (0,)), ((), ())),
        preferred_element_type=jnp.float32)
    tot = cum[NBINS - 1:NBINS, :]
    rcum = tot - cum + rhs
    w1 = cum[0:NBINS - 1, 0:1]
    s1 = cum[0:NBINS - 1, 1:2]
    w2 = rcum[1:NBINS, 0:1]
    s2 = rcum[1:NBINS, 1:2]
    m1 = s1 / w1
    m2 = s2 / w2
    d = m1 - m2
    var12 = w1 * w2 * d * d
    bstar = jnp.argmax(var12).astype(jnp.float32)
    thresh = mn + (bstar + 0.5) * (span / NBINS)

    t = t_ref[0]
    maskf = jnp.where(p > thresh, 1.0, 0.0)
    inter = jnp.sum(maskf * t)
    nb = jnp.sum(maskf)
    st = jnp.sum(t)
    union = nb + st - inter
    iou = (inter + 1.0) / (union + 1.0)
    o_ref[...] = jnp.broadcast_to(iou, (1, 1, 128))


def kernel(output, target):
    B = output.shape[0]
    x = output.reshape(B, H, W)
    t = target.reshape(B, H, W)
    res = pl.pallas_call(
        _iou_kernel,
        grid=(B,),
        in_specs=[
            pl.BlockSpec((1, H, W), lambda s: (s, 0, 0)),
            pl.BlockSpec((1, H, W), lambda s: (s, 0, 0)),
        ],
        out_specs=pl.BlockSpec((1, 1, 128), lambda s: (s, 0, 0)),
        out_shape=jax.ShapeDtypeStruct((B, 1, 128), jnp.float32),
        scratch_shapes=[
            pltpu.VMEM((H, W), jnp.float32),
            pltpu.VMEM((NBINS, W), jnp.float32),
        ],
        compiler_params=pltpu.CompilerParams(
            dimension_semantics=("parallel",),
        ),
    )(x, t)
    return jnp.mean(res[:, 0, 0])


# speedup vs baseline: 16.9178x; 4.3399x over previous
import jax
import jax.numpy as jnp
from jax.experimental import pallas as pl
from jax.experimental.pallas import tpu as pltpu

NBINS = 256
H = 512
W = 512
CHUNK = 16
NGROUPS = 16
NSTRIPS = 4


def _iou_kernel(x_ref, t_ref, o_ref, p_ref, acc_ref):
    p_ref[...] = jax.nn.sigmoid(x_ref[0])
    p = p_ref[...]
    mn = jnp.min(p)
    mx = jnp.max(p)
    span = mx - mn
    scale = jnp.where(span > 0, jnp.float32(NBINS) / span, jnp.float32(0.0))

    acc_ref[...] = jnp.zeros_like(acc_ref)
    binv = [
        (jax.lax.broadcasted_iota(jnp.int32, (16, 128), 0) + 16 * g
         ).astype(jnp.bfloat16)
        for g in range(NGROUPS)
    ]
    one = jnp.bfloat16(1.0)

    for ls in range(NSTRIPS):
        lo = ls * 128
        hi = lo + 128

        def hbody(c, carry):
            rows = p_ref[pl.ds(c * CHUNK, CHUNK), lo:hi]
            idxf = jnp.clip(jnp.floor((rows - mn) * scale), 0.0,
                            float(NBINS - 1))
            parts = [jnp.zeros((16, 128), jnp.bfloat16)
                     for _ in range(NGROUPS)]
            for r in range(CHUNK):
                rowb = jnp.broadcast_to(idxf[r:r + 1, :], (16, 128)).astype(
                    jnp.bfloat16)
                for g in range(NGROUPS):
                    parts[g] = jnp.where(rowb == binv[g], parts[g] + one,
                                         parts[g])
            for g in range(NGROUPS):
                acc_ref[g * 16:(g + 1) * 16, lo:hi] += parts[g].astype(
                    jnp.float32)
            return carry

        jax.lax.fori_loop(0, H // CHUNK, hbody, 0)

    hist = jnp.sum(acc_ref[...], axis=1, keepdims=True)
    bins = jax.lax.broadcasted_iota(jnp.int32, (NBINS, 1), 0).astype(
        jnp.float32)
    centers = mn + (bins + 0.5) * (span / NBINS)
    rhs = jnp.concatenate([hist, hist * centers], axis=1)
    ri = jax.lax.broadcasted_iota(jnp.int32, (NBINS, NBINS), 0)
    ci = jax.lax.broadcasted_iota(jnp.int32, (NBINS, NBINS), 1)
    ltri = jnp.where(ci <= ri, 1.0, 0.0)
    cum = jax.lax.dot_general(
        ltri, rhs, (((1,), (0,)), ((), ())),
        preferred_element_type=jnp.float32)
    tot = cum[NBINS - 1:NBINS, :]
    rcum = tot - cum + rhs
    w1 = cum[0:NBINS - 1, 0:1]
    s1 = cum[0:NBINS - 1, 1:2]
    w2 = rcum[1:NBINS, 0:1]
    s2 = rcum[1:NBINS, 1:2]
    m1 = s1 / w1
    m2 = s2 / w2
    d = m1 - m2
    var12 = w1 * w2 * d * d
    bstar = jnp.argmax(var12).astype(jnp.float32)
    thresh = mn + (bstar + 0.5) * (span / NBINS)

    t = t_ref[0]
    maskf = jnp.where(p > thresh, 1.0, 0.0)
    inter = jnp.sum(maskf * t)
    nb = jnp.sum(maskf)
    st = jnp.sum(t)
    union = nb + st - inter
    iou = (inter + 1.0) / (union + 1.0)
    o_ref[...] = jnp.broadcast_to(iou, (1, 1, 128))


def kernel(output, target):
    B = output.shape[0]
    x = output.reshape(B, H, W)
    t = target.reshape(B, H, W)
    res = pl.pallas_call(
        _iou_kernel,
        grid=(B,),
        in_specs=[
            pl.BlockSpec((1, H, W), lambda s: (s, 0, 0)),
            pl.BlockSpec((1, H, W), lambda s: (s, 0, 0)),
        ],
        out_specs=pl.BlockSpec((1, 1, 128), lambda s: (s, 0, 0)),
        out_shape=jax.ShapeDtypeStruct((B, 1, 128), jnp.float32),
        scratch_shapes=[
            pltpu.VMEM((H, W), jnp.float32),
            pltpu.VMEM((NBINS, W), jnp.float32),
        ],
        compiler_params=pltpu.CompilerParams(
            dimension_semantics=("parallel",),
        ),
    )(x, t)
    return jnp.mean(res[:, 0, 0])


# speedup vs baseline: 17.6367x; 1.0425x over previous
import jax
import jax.numpy as jnp
from jax.experimental import pallas as pl
from jax.experimental.pallas import tpu as pltpu

NBINS = 256
H = 512
W = 512
CHUNK = 32
NGROUPS = 16
NSTRIPS = 4


def _iou_kernel(x_ref, t_ref, o_ref, p_ref, acc_ref):
    p_ref[...] = jax.nn.sigmoid(x_ref[0])
    p = p_ref[...]
    mn = jnp.min(p)
    mx = jnp.max(p)
    span = mx - mn
    scale = jnp.where(span > 0, jnp.float32(NBINS) / span, jnp.float32(0.0))

    acc_ref[...] = jnp.zeros_like(acc_ref)
    binv = [
        (jax.lax.broadcasted_iota(jnp.int32, (16, 128), 0) + 16 * g
         ).astype(jnp.bfloat16)
        for g in range(NGROUPS)
    ]
    one = jnp.bfloat16(1.0)

    for ls in range(NSTRIPS):
        lo = ls * 128
        hi = lo + 128

        def hbody(c, carry):
            rows = p_ref[pl.ds(c * CHUNK, CHUNK), lo:hi]
            idxf = jnp.clip(jnp.floor((rows - mn) * scale), 0.0,
                            float(NBINS - 1))
            parts = [jnp.zeros((16, 128), jnp.bfloat16)
                     for _ in range(NGROUPS)]
            for r in range(CHUNK):
                rowb = jnp.broadcast_to(idxf[r:r + 1, :], (16, 128)).astype(
                    jnp.bfloat16)
                for g in range(NGROUPS):
                    parts[g] = jnp.where(rowb == binv[g], parts[g] + one,
                                         parts[g])
            for g in range(NGROUPS):
                acc_ref[g * 16:(g + 1) * 16, lo:hi] += parts[g].astype(
                    jnp.float32)
            return carry

        jax.lax.fori_loop(0, H // CHUNK, hbody, 0)

    hist = jnp.sum(acc_ref[...], axis=1, keepdims=True)
    bins = jax.lax.broadcasted_iota(jnp.int32, (NBINS, 1), 0).astype(
        jnp.float32)
    centers = mn + (bins + 0.5) * (span / NBINS)
    rhs = jnp.concatenate([hist, hist * centers], axis=1)
    ri = jax.lax.broadcasted_iota(jnp.int32, (NBINS, NBINS), 0)
    ci = jax.lax.broadcasted_iota(jnp.int32, (NBINS, NBINS), 1)
    ltri = jnp.where(ci <= ri, 1.0, 0.0)
    cum = jax.lax.dot_general(
        ltri, rhs, (((1,), (0,)), ((), ())),
        preferred_element_type=jnp.float32)
    tot = cum[NBINS - 1:NBINS, :]
    rcum = tot - cum + rhs
    w1 = cum[0:NBINS - 1, 0:1]
    s1 = cum[0:NBINS - 1, 1:2]
    w2 = rcum[1:NBINS, 0:1]
    s2 = rcum[1:NBINS, 1:2]
    m1 = s1 / w1
    m2 = s2 / w2
    d = m1 - m2
    var12 = w1 * w2 * d * d
    bstar = jnp.argmax(var12).astype(jnp.float32)
    thresh = mn + (bstar + 0.5) * (span / NBINS)

    t = t_ref[0]
    maskf = jnp.where(p > thresh, 1.0, 0.0)
    inter = jnp.sum(maskf * t)
    nb = jnp.sum(maskf)
    st = jnp.sum(t)
    union = nb + st - inter
    iou = (inter + 1.0) / (union + 1.0)
    o_ref[...] = jnp.broadcast_to(iou, (1, 1, 128))


def kernel(output, target):
    B = output.shape[0]
    x = output.reshape(B, H, W)
    t = target.reshape(B, H, W)
    res = pl.pallas_call(
        _iou_kernel,
        grid=(B,),
        in_specs=[
            pl.BlockSpec((1, H, W), lambda s: (s, 0, 0)),
            pl.BlockSpec((1, H, W), lambda s: (s, 0, 0)),
        ],
        out_specs=pl.BlockSpec((1, 1, 128), lambda s: (s, 0, 0)),
        out_shape=jax.ShapeDtypeStruct((B, 1, 128), jnp.float32),
        scratch_shapes=[
            pltpu.VMEM((H, W), jnp.float32),
            pltpu.VMEM((NBINS, W), jnp.float32),
        ],
        compiler_params=pltpu.CompilerParams(
            dimension_semantics=("parallel",),
        ),
    )(x, t)
    return jnp.mean(res[:, 0, 0])


# speedup vs baseline: 31.1316x; 1.7652x over previous
import jax
import jax.numpy as jnp
from jax.experimental import pallas as pl
from jax.experimental.pallas import tpu as pltpu

NBINS = 256
N = 512 * 512
GS = 16
NG = 64 // GS
CHUNK = 8192


def _thresh_kernel(x_ref, o_ref, acc_ref):
    x2 = x_ref[0]
    mn = jnp.min(x2, axis=1, keepdims=True)
    mx = jnp.max(x2, axis=1, keepdims=True)
    mnp = jax.nn.sigmoid(mn)
    mxp = jax.nn.sigmoid(mx)
    span = mxp - mnp
    scale = jnp.where(span > 0, jnp.float32(NBINS) / span, jnp.float32(0.0))

    acc_ref[...] = jnp.zeros_like(acc_ref)
    bins16 = jax.lax.broadcasted_iota(jnp.int32, (1, 16, 1), 1).astype(
        jnp.bfloat16)
    one = jnp.bfloat16(1.0)
    zero = jnp.bfloat16(0.0)
    for c in range(N // CHUNK):
        xc = x_ref[0, :, c * CHUNK:(c + 1) * CHUNK]
        pc = jax.nn.sigmoid(xc)
        idx = jnp.clip(jnp.floor((pc - mnp) * scale), 0.0, float(NBINS - 1))
        hi = jnp.floor(idx * jnp.float32(1.0 / 16.0))
        lo = idx - 16.0 * hi
        hib = hi.astype(jnp.bfloat16)
        lob = lo.astype(jnp.bfloat16)
        o_l = jnp.where(hib[:, None, :] == bins16, one, zero).reshape(
            16 * 16, CHUNK)
        o_r = jnp.where(lob[:, None, :] == bins16, one, zero).reshape(
            16 * 16, CHUNK)
        acc_ref[...] += jax.lax.dot_general(
            o_l, o_r, (((1,), (1,)), ((), ())),
            preferred_element_type=jnp.float32)

    ri = jax.lax.broadcasted_iota(jnp.int32, (16, 16), 0)
    ci = jax.lax.broadcasted_iota(jnp.int32, (16, 16), 1)
    upper_incl = jnp.where(ri <= ci, 1.0, 0.0)
    lower_strict = jnp.where(ci < ri, 1.0, 0.0)
    lex = (16.0 * ri.astype(jnp.float32) + ci.astype(jnp.float32))
    lane_i = jax.lax.broadcasted_iota(jnp.int32, (1, 128), 1)
    cfull = acc_ref[...]
    th_vec = jnp.zeros((1, 128), jnp.float32)
    for s in range(GS):
        hs = cfull[16 * s:16 * s + 16, 16 * s:16 * s + 16]
        mnp_s = mnp[s:s + 1, :]
        span_s = span[s:s + 1, :]
        centers = mnp_s + (lex + 0.5) * (span_s / NBINS)
        hc = hs * centers
        rowcum = jax.lax.dot_general(
            hs, upper_incl, (((1,), (0,)), ((), ())),
            preferred_element_type=jnp.float32)
        rowcum_c = jax.lax.dot_general(
            hc, upper_incl, (((1,), (0,)), ((), ())),
            preferred_element_type=jnp.float32)
        p_cnt = jax.lax.dot_general(
            lower_strict, rowcum[:, 15:16], (((1,), (0,)), ((), ())),
            preferred_element_type=jnp.float32)
        p_c = jax.lax.dot_general(
            lower_strict, rowcum_c[:, 15:16], (((1,), (0,)), ((), ())),
            preferred_element_type=jnp.float32)
        w1 = rowcum + p_cnt
        s1 = rowcum_c + p_c
        tot = w1[15:16, 15:16]
        tots = s1[15:16, 15:16]
        w2n = tot - w1
        s2n = tots - s1
        m1 = s1 / w1
        m2 = s2n / w2n
        d = m1 - m2
        var = w1 * w2n * d * d
        var = jnp.where(lex >= float(NBINS - 1), -1.0, var)
        vmaxv = jnp.max(var)
        lexm = jnp.where(var == vmaxv, lex, jnp.float32(NBINS))
        bstar = jnp.min(lexm)
        th_s = mnp_s + (bstar + 0.5) * (span_s / NBINS)
        th_vec = th_vec + jnp.where(lane_i == s,
                                    jnp.broadcast_to(th_s, (1, 128)), 0.0)
    o_ref[...] = th_vec[None]


def _iou_kernel(x_ref, t_ref, th_ref, o_ref):
    th = th_ref[0, 0, 0]
    p = jax.nn.sigmoid(x_ref[0])
    t = t_ref[0]
    maskf = jnp.where(p > th, 1.0, 0.0)
    inter = jnp.sum(maskf * t)
    nb = jnp.sum(maskf)
    st = jnp.sum(t)
    union = nb + st - inter
    iou = (inter + 1.0) / (union + 1.0)
    o_ref[...] = jnp.broadcast_to(iou, (1, 1, 128))


def kernel(output, target):
    B = output.shape[0]
    xg = output.reshape(NG, GS, N)
    th4 = pl.pallas_call(
        _thresh_kernel,
        grid=(NG,),
        in_specs=[pl.BlockSpec((1, GS, N), lambda g: (g, 0, 0))],
        out_specs=pl.BlockSpec((1, 1, 128), lambda g: (g, 0, 0)),
        out_shape=jax.ShapeDtypeStruct((NG, 1, 128), jnp.float32),
        scratch_shapes=[pltpu.VMEM((NBINS, NBINS), jnp.float32)],
        compiler_params=pltpu.CompilerParams(
            dimension_semantics=("parallel",),
        ),
    )(xg)
    thr = jnp.broadcast_to(th4[:, 0, :GS].reshape(B)[:, None, None],
                           (B, 1, 128))

    x3 = output.reshape(B, 16, N // 16)
    t3 = target.reshape(B, 16, N // 16)
    res = pl.pallas_call(
        _iou_kernel,
        grid=(B,),
        in_specs=[
            pl.BlockSpec((1, 16, N // 16), lambda s: (s, 0, 0)),
            pl.BlockSpec((1, 16, N // 16), lambda s: (s, 0, 0)),
            pl.BlockSpec((1, 1, 128), lambda s: (s, 0, 0)),
        ],
        out_specs=pl.BlockSpec((1, 1, 128), lambda s: (s, 0, 0)),
        out_shape=jax.ShapeDtypeStruct((B, 1, 128), jnp.float32),
        compiler_params=pltpu.CompilerParams(
            dimension_semantics=("parallel",),
        ),
    )(x3, t3, thr)
    return jnp.mean(res[:, 0, 0])
